# baseline (speedup 1.0000x reference)
import functools

import jax
import jax.numpy as jnp
from jax import lax
from jax.experimental import pallas as pl
from jax.experimental.pallas import tpu as pltpu
from jax.experimental.pallas import tpu_sc as plsc

_NC = 2
_NS = 16
_NW = _NC * _NS
_GB = 80

_N = 50000
_D = 128
_CHUNK = 32
_NCHUNK = _D // _CHUNK
_ROWS_PER_TILE = _N // _NS

_UNITS = (
    (0, 0, 0), (0, 1, 0), (0, 2, 0), (1, 0, 0), (1, 1, 0),
    (0, 3, 1), (2, 0, 1), (2, 1, 1), (2, 2, 1), (2, 3, 1), (1, 2, 1), (1, 3, 1),
)


def _tc_block_call(xs, Ws, b1, W2, b2, gamma, beta, resid, bs):
    R = xs[0].shape[0]
    H = W2.shape[0]
    O = W2.shape[1]
    nx = len(xs)
    ln = gamma is not None
    has_res = resid is not None

    def body(*refs):
        out_r = refs[-1]
        xr = refs[:nx]
        wr = refs[nx:2 * nx]
        k = 2 * nx
        b1r, w2r, b2r = refs[k], refs[k + 1], refs[k + 2]
        k += 3
        h = jnp.dot(xr[0][...], wr[0][...], preferred_element_type=jnp.float32)
        for t in range(1, nx):
            h = h + jnp.dot(xr[t][...], wr[t][...],
                            preferred_element_type=jnp.float32)
        h = jnp.maximum(h + b1r[...], 0.0)
        y = jnp.dot(h, w2r[...], preferred_element_type=jnp.float32) + b2r[...]
        if ln:
            gr, btr = refs[k], refs[k + 1]
            k += 2
            mu = jnp.mean(y, axis=-1, keepdims=True)
            d = y - mu
            var = jnp.mean(d * d, axis=-1, keepdims=True)
            y = gr[...] * (d / jnp.sqrt(var + 1e-5)) + btr[...]
        if has_res:
            y = y + refs[k][...]
        out_r[...] = y

    in_specs = (
        [pl.BlockSpec((bs, x.shape[1]), lambda i: (i, 0)) for x in xs]
        + [pl.BlockSpec(W.shape, lambda i: (0, 0)) for W in Ws]
        + [pl.BlockSpec((1, H), lambda i: (0, 0)),
           pl.BlockSpec((H, O), lambda i: (0, 0)),
           pl.BlockSpec((1, O), lambda i: (0, 0))]
    )
    args = list(xs) + list(Ws) + [b1.reshape(1, -1), W2, b2.reshape(1, -1)]
    if ln:
        in_specs += [pl.BlockSpec((1, O), lambda i: (0, 0))] * 2
        args += [gamma.reshape(1, -1), beta.reshape(1, -1)]
    if has_res:
        in_specs += [pl.BlockSpec((bs, O), lambda i: (i, 0))]
        args += [resid]
    return pl.pallas_call(
        body,
        grid=(R // bs,),
        in_specs=in_specs,
        out_specs=pl.BlockSpec((bs, O), lambda i: (i, 0)),
        out_shape=jax.ShapeDtypeStruct((R, O), jnp.float32),
    )(*args)


def _apply_block(blk, xs, resid=None, bs=2000):
    mlp = blk["mlp"]
    W1, b1 = mlp[0]["W"], mlp[0]["b"]
    W2, b2 = mlp[1]["W"], mlp[1]["b"]
    Ws, off = [], 0
    for xx in xs:
        w = xx.shape[1]
        Ws.append(lax.slice_in_dim(W1, off, off + w, axis=0).astype(xx.dtype))
        off += w
    return _tc_block_call(xs, Ws, b1, W2, b2, blk["gamma"], blk["beta"],
                          resid, bs)


def _apply_mlp(mlp, x, bs=2000):
    W1, b1 = mlp[0]["W"], mlp[0]["b"]
    W2, b2 = mlp[1]["W"], mlp[1]["b"]
    O = W2.shape[1]
    W2p = jnp.pad(W2, ((0, 0), (0, _D - O)))
    b2p = jnp.pad(b2, (0, _D - O))
    out = _tc_block_call([x], [W1], b1, W2p, b2p, None, None, None, bs)
    return out[:, :O]


_GNB = 6
_GBG = 128


def _sc_gather(x, idx_list):
    Es = [int(i.shape[0]) for i in idx_list]
    D = int(x.shape[1])
    dt = x.dtype
    na = len(Es)
    mesh = plsc.VectorSubcoreMesh(core_axis_name="c", subcore_axis_name="s")

    def body(*refs):
        x_hbm = refs[0]
        idx_refs = refs[1:1 + na]
        out_refs = refs[1 + na:1 + 2 * na]
        rest = refs[1 + 2 * na:]
        idx_bufs = rest[:_GNB]
        row_bufs = rest[_GNB:2 * _GNB]
        isems = rest[2 * _GNB:3 * _GNB]
        gsems = rest[3 * _GNB:4 * _GNB]
        wsems = rest[4 * _GNB:5 * _GNB]
        wid = lax.axis_index("s") * _NC + lax.axis_index("c")
        for a in range(na):
            nb = -(-Es[a] // _GBG)
            cnt = -(-nb // _NW)
            cnt = -(-cnt // _GNB) * _GNB
            span = nb - cnt
            start = (wid * span) // (_NW - 1)
            ih = idx_refs[a]
            oh = out_refs[a]

            def bod(g, carry, ih=ih, oh=oh, start=start, E=Es[a]):
                k0 = start + g * _GNB

                def bs(b):
                    return pl.multiple_of(
                        jnp.minimum((k0 + b) * _GBG, E - _GBG), 8)

                for b in range(_GNB):
                    pltpu.async_copy(ih.at[pl.ds(bs(b), _GBG)], idx_bufs[b],
                                     isems[b])
                for b in range(_GNB):
                    pltpu.make_async_copy(ih.at[pl.ds(bs(b), _GBG)],
                                          idx_bufs[b], isems[b]).wait()
                    pltpu.async_copy(x_hbm.at[idx_bufs[b]], row_bufs[b],
                                     gsems[b])
                for b in range(_GNB):
                    pltpu.make_async_copy(x_hbm.at[idx_bufs[b]], row_bufs[b],
                                          gsems[b]).wait()
                    pltpu.async_copy(row_bufs[b], oh.at[pl.ds(bs(b), _GBG)],
                                     wsems[b])
                for b in range(_GNB):
                    pltpu.make_async_copy(row_bufs[b],
                                          oh.at[pl.ds(bs(b), _GBG)],
                                          wsems[b]).wait()
                return carry

            lax.fori_loop(0, cnt // _GNB, bod, 0)

    f = pl.kernel(
        body,
        out_type=[jax.ShapeDtypeStruct((E, D), dt) for E in Es],
        mesh=mesh,
        compiler_params=pltpu.CompilerParams(use_tc_tiling_on_sc=False),
        scratch_types=(
            [pltpu.VMEM((_GBG,), jnp.int32) for _ in range(_GNB)]
            + [pltpu.VMEM((_GBG, D), dt) for _ in range(_GNB)]
            + [pltpu.SemaphoreType.DMA for _ in range(3 * _GNB)]
        ),
    )
    return f(x, *idx_list)


_SNB = 6


_SNB = 6


def _sc_scatter_one(enew, dst):
    units = ((0, 0), (1, 0), (2, 1), (3, 1))
    E = int(enew.shape[0])
    mesh = plsc.VectorSubcoreMesh(core_axis_name="c", subcore_axis_name="s")

    def body(*refs):
        eh, dh, ah, shared, zbuf = refs[:5]
        rest = refs[5:]
        idx_bufs = rest[:_SNB]
        val_bufs = rest[_SNB:2 * _SNB]
        isems = rest[2 * _SNB:3 * _SNB]
        vsems = rest[3 * _SNB:4 * _SNB]
        ssems = rest[4 * _SNB:5 * _SNB]
        cid = lax.axis_index("c")
        sid = lax.axis_index("s")

        def zinit(r, carry):
            zbuf[r, pl.ds(0, 16)] = jnp.zeros((16,), jnp.float32)
            zbuf[r, pl.ds(16, 16)] = jnp.zeros((16,), jnp.float32)
            return carry

        lax.fori_loop(0, 125, zinit, 0)

        r0 = sid * _ROWS_PER_TILE
        nb = E // _GB
        lo = (nb * sid) // _NS
        hi = (nb * (sid + 1)) // _NS
        for (c, core) in units:
            @pl.when(cid == core)
            def _unit(c=c):
                for j in range(_ROWS_PER_TILE // 125):
                    pltpu.sync_copy(zbuf, shared.at[pl.ds(r0 + j * 125, 125)])
                plsc.subcore_barrier()

                def esl(k, c=c):
                    base = pl.multiple_of(k * _GB, 8)
                    return eh.at[pl.ds(base, _GB),
                                 pl.ds(c * _CHUNK, _CHUNK)]

                def dsl(k):
                    base = pl.multiple_of(k * _GB, 8)
                    return dh.at[pl.ds(base, _GB)]

                def bod(g, carry):
                    k0 = lo + g * _SNB
                    for b in range(_SNB):
                        k = k0 + b

                        @pl.when(k < hi)
                        def _fire(b=b, k=k):
                            pltpu.async_copy(dsl(k), idx_bufs[b], isems[b])
                            pltpu.async_copy(esl(k), val_bufs[b], vsems[b])
                    for b in range(_SNB):
                        k = k0 + b

                        @pl.when(k < hi)
                        def _scat(b=b, k=k):
                            pltpu.make_async_copy(dsl(k), idx_bufs[b],
                                                  isems[b]).wait()
                            pltpu.make_async_copy(esl(k), val_bufs[b],
                                                  vsems[b]).wait()
                            pltpu.async_copy(val_bufs[b],
                                             shared.at[idx_bufs[b]],
                                             ssems[b], add=True)
                    for b in range(_SNB):
                        k = k0 + b

                        @pl.when(k < hi)
                        def _drain(b=b):
                            pltpu.make_async_copy(val_bufs[b],
                                                  shared.at[idx_bufs[b]],
                                                  ssems[b]).wait()
                    return carry

                nch = (hi - lo + _SNB - 1) // _SNB
                lax.fori_loop(0, nch, bod, 0)
                plsc.subcore_barrier()
                pltpu.sync_copy(
                    shared.at[pl.ds(r0, _ROWS_PER_TILE)],
                    ah.at[pl.ds(r0, _ROWS_PER_TILE),
                          pl.ds(c * _CHUNK, _CHUNK)])
                plsc.subcore_barrier()

    f = pl.kernel(
        body,
        out_type=jax.ShapeDtypeStruct((_N, _D), jnp.float32),
        mesh=mesh,
        compiler_params=pltpu.CompilerParams(use_tc_tiling_on_sc=False),
        scratch_types=(
            [pltpu.VMEM_SHARED((_N, _CHUNK), jnp.float32),
             pltpu.VMEM((125, _CHUNK), jnp.float32)]
            + [pltpu.VMEM((_GB,), jnp.int32) for _ in range(_SNB)]
            + [pltpu.VMEM((_GB, _CHUNK), jnp.float32) for _ in range(_SNB)]
            + [pltpu.SemaphoreType.DMA for _ in range(3 * _SNB)]
        ),
    )
    return f(enew, dst)


def kernel(node_x, body_edge_attr, body_edge_index, cable_edge_attr,
           cable_edge_index, contact_edge_attr, contact_edge_index, params):
    p = params
    x = _apply_block(p["enc_node"], [node_x])
    be = _apply_block(p["enc_body"], [body_edge_attr])
    ce = _apply_block(p["enc_cable"], [cable_edge_attr])
    cte = _apply_block(p["enc_contact"], [contact_edge_attr])

    sb, db = body_edge_index[0], body_edge_index[1]
    sc_, dc = cable_edge_index[0], cable_edge_index[1]
    sct, dct = contact_edge_index[0], contact_edge_index[1]

    for st in p["proc"]:
        xi_b, xj_b = _sc_gather(x, [db, sb])
        xi_c, xj_c = _sc_gather(x, [dc, sc_])
        xi_ct, xj_ct = _sc_gather(x, [dct, sct])
        be = _apply_block(st["body"], [xi_b, xj_b, be], resid=be)
        agg_b = _sc_scatter_one(be, db)
        ce = _apply_block(st["cable"], [xi_c, xj_c, ce], resid=ce)
        agg_c = _sc_scatter_one(ce, dc)
        cte = _apply_block(st["contact"], [xi_ct, xj_ct, cte], resid=cte)
        agg_ct = _sc_scatter_one(cte, dct)
        x = _apply_block(st["update"], [x, agg_b, agg_c, agg_ct], resid=x)

    dec = _apply_mlp(p["dec_node"], x)
    cdec = _apply_mlp(p["dec_cable"], ce)
    return (dec, cdec)


# speedup vs baseline: 1.0150x; 1.0150x over previous
import functools

import jax
import jax.numpy as jnp
from jax import lax
from jax.experimental import pallas as pl
from jax.experimental.pallas import tpu as pltpu
from jax.experimental.pallas import tpu_sc as plsc

_NC = 2
_NS = 16
_NW = _NC * _NS
_GB = 80

_N = 50000
_D = 128
_CHUNK = 32
_NCHUNK = _D // _CHUNK
_ROWS_PER_TILE = _N // _NS

_UNITS = (
    (0, 0, 0), (0, 1, 0), (0, 2, 0), (1, 0, 0), (1, 1, 0),
    (0, 3, 1), (2, 0, 1), (2, 1, 1), (2, 2, 1), (2, 3, 1), (1, 2, 1), (1, 3, 1),
)


def _tc_block_call(xs, Ws, b1, W2, b2, gamma, beta, resid, bs):
    R = xs[0].shape[0]
    H = W2.shape[0]
    O = W2.shape[1]
    nx = len(xs)
    ln = gamma is not None
    has_res = resid is not None

    def body(*refs):
        out_r = refs[-1]
        xr = refs[:nx]
        wr = refs[nx:2 * nx]
        k = 2 * nx
        b1r, w2r, b2r = refs[k], refs[k + 1], refs[k + 2]
        k += 3
        h = jnp.dot(xr[0][...], wr[0][...], preferred_element_type=jnp.float32)
        for t in range(1, nx):
            h = h + jnp.dot(xr[t][...], wr[t][...],
                            preferred_element_type=jnp.float32)
        h = jnp.maximum(h + b1r[...], 0.0)
        y = jnp.dot(h, w2r[...], preferred_element_type=jnp.float32) + b2r[...]
        if ln:
            gr, btr = refs[k], refs[k + 1]
            k += 2
            mu = jnp.mean(y, axis=-1, keepdims=True)
            d = y - mu
            var = jnp.mean(d * d, axis=-1, keepdims=True)
            y = gr[...] * (d / jnp.sqrt(var + 1e-5)) + btr[...]
        if has_res:
            y = y + refs[k][...]
        out_r[...] = y

    in_specs = (
        [pl.BlockSpec((bs, x.shape[1]), lambda i: (i, 0)) for x in xs]
        + [pl.BlockSpec(W.shape, lambda i: (0, 0)) for W in Ws]
        + [pl.BlockSpec((1, H), lambda i: (0, 0)),
           pl.BlockSpec((H, O), lambda i: (0, 0)),
           pl.BlockSpec((1, O), lambda i: (0, 0))]
    )
    args = list(xs) + list(Ws) + [b1.reshape(1, -1), W2, b2.reshape(1, -1)]
    if ln:
        in_specs += [pl.BlockSpec((1, O), lambda i: (0, 0))] * 2
        args += [gamma.reshape(1, -1), beta.reshape(1, -1)]
    if has_res:
        in_specs += [pl.BlockSpec((bs, O), lambda i: (i, 0))]
        args += [resid]
    return pl.pallas_call(
        body,
        grid=(R // bs,),
        in_specs=in_specs,
        out_specs=pl.BlockSpec((bs, O), lambda i: (i, 0)),
        out_shape=jax.ShapeDtypeStruct((R, O), jnp.float32),
    )(*args)


def _apply_block(blk, xs, resid=None, bs=2000):
    mlp = blk["mlp"]
    W1, b1 = mlp[0]["W"], mlp[0]["b"]
    W2, b2 = mlp[1]["W"], mlp[1]["b"]
    Ws, off = [], 0
    for xx in xs:
        w = xx.shape[1]
        Ws.append(lax.slice_in_dim(W1, off, off + w, axis=0).astype(xx.dtype))
        off += w
    return _tc_block_call(xs, Ws, b1, W2, b2, blk["gamma"], blk["beta"],
                          resid, bs)


def _apply_mlp(mlp, x, bs=2000):
    W1, b1 = mlp[0]["W"], mlp[0]["b"]
    W2, b2 = mlp[1]["W"], mlp[1]["b"]
    O = W2.shape[1]
    W2p = jnp.pad(W2, ((0, 0), (0, _D - O)))
    b2p = jnp.pad(b2, (0, _D - O))
    out = _tc_block_call([x], [W1], b1, W2p, b2p, None, None, None, bs)
    return out[:, :O]


_GNB = 8
_GBG = 80


def _sc_gather(x, idx_list):
    Es = [int(i.shape[0]) for i in idx_list]
    D = int(x.shape[1])
    dt = x.dtype
    na = len(Es)
    mesh = plsc.VectorSubcoreMesh(core_axis_name="c", subcore_axis_name="s")

    def body(*refs):
        x_hbm = refs[0]
        idx_refs = refs[1:1 + na]
        out_refs = refs[1 + na:1 + 2 * na]
        rest = refs[1 + 2 * na:]
        idx_bufs = rest[:_GNB]
        row_bufs = rest[_GNB:2 * _GNB]
        isems = rest[2 * _GNB:3 * _GNB]
        gsems = rest[3 * _GNB:4 * _GNB]
        wsems = rest[4 * _GNB:5 * _GNB]
        wid = lax.axis_index("s") * _NC + lax.axis_index("c")
        for a in range(na):
            nb = -(-Es[a] // _GBG)
            cnt = -(-nb // _NW)
            cnt = -(-cnt // _GNB) * _GNB
            span = nb - cnt
            start = (wid * span) // (_NW - 1)
            ih = idx_refs[a]
            oh = out_refs[a]

            def bod(g, carry, ih=ih, oh=oh, start=start, E=Es[a]):
                k0 = start + g * _GNB

                def bs(b):
                    return pl.multiple_of(
                        jnp.minimum((k0 + b) * _GBG, E - _GBG), 8)

                for b in range(_GNB):
                    pltpu.async_copy(ih.at[pl.ds(bs(b), _GBG)], idx_bufs[b],
                                     isems[b])
                for b in range(_GNB):
                    pltpu.make_async_copy(ih.at[pl.ds(bs(b), _GBG)],
                                          idx_bufs[b], isems[b]).wait()
                    pltpu.async_copy(x_hbm.at[idx_bufs[b]], row_bufs[b],
                                     gsems[b])
                for b in range(_GNB):
                    pltpu.make_async_copy(x_hbm.at[idx_bufs[b]], row_bufs[b],
                                          gsems[b]).wait()
                    pltpu.async_copy(row_bufs[b], oh.at[pl.ds(bs(b), _GBG)],
                                     wsems[b])
                for b in range(_GNB):
                    pltpu.make_async_copy(row_bufs[b],
                                          oh.at[pl.ds(bs(b), _GBG)],
                                          wsems[b]).wait()
                return carry

            lax.fori_loop(0, cnt // _GNB, bod, 0)

    f = pl.kernel(
        body,
        out_type=[jax.ShapeDtypeStruct((E, D), dt) for E in Es],
        mesh=mesh,
        compiler_params=pltpu.CompilerParams(use_tc_tiling_on_sc=False),
        scratch_types=(
            [pltpu.VMEM((_GBG,), jnp.int32) for _ in range(_GNB)]
            + [pltpu.VMEM((_GBG, D), dt) for _ in range(_GNB)]
            + [pltpu.SemaphoreType.DMA for _ in range(3 * _GNB)]
        ),
    )
    return f(x, *idx_list)


_SNB = 6


_SNB = 6


def _sc_scatter_one(enew, dst):
    units = ((0, 0), (1, 0), (2, 1), (3, 1))
    E = int(enew.shape[0])
    mesh = plsc.VectorSubcoreMesh(core_axis_name="c", subcore_axis_name="s")

    def body(*refs):
        eh, dh, ah, shared, zbuf = refs[:5]
        rest = refs[5:]
        idx_bufs = rest[:_SNB]
        val_bufs = rest[_SNB:2 * _SNB]
        isems = rest[2 * _SNB:3 * _SNB]
        vsems = rest[3 * _SNB:4 * _SNB]
        ssems = rest[4 * _SNB:5 * _SNB]
        cid = lax.axis_index("c")
        sid = lax.axis_index("s")

        def zinit(r, carry):
            zbuf[r, pl.ds(0, 16)] = jnp.zeros((16,), jnp.float32)
            zbuf[r, pl.ds(16, 16)] = jnp.zeros((16,), jnp.float32)
            return carry

        lax.fori_loop(0, 125, zinit, 0)

        r0 = sid * _ROWS_PER_TILE
        nb = E // _GB
        lo = (nb * sid) // _NS
        hi = (nb * (sid + 1)) // _NS
        for (c, core) in units:
            @pl.when(cid == core)
            def _unit(c=c):
                for j in range(_ROWS_PER_TILE // 125):
                    pltpu.sync_copy(zbuf, shared.at[pl.ds(r0 + j * 125, 125)])
                plsc.subcore_barrier()

                def esl(k, c=c):
                    base = pl.multiple_of(k * _GB, 8)
                    return eh.at[pl.ds(base, _GB),
                                 pl.ds(c * _CHUNK, _CHUNK)]

                def dsl(k):
                    base = pl.multiple_of(k * _GB, 8)
                    return dh.at[pl.ds(base, _GB)]

                def bod(g, carry):
                    k0 = lo + g * _SNB
                    for b in range(_SNB):
                        k = k0 + b

                        @pl.when(k < hi)
                        def _fire(b=b, k=k):
                            pltpu.async_copy(dsl(k), idx_bufs[b], isems[b])
                            pltpu.async_copy(esl(k), val_bufs[b], vsems[b])
                    for b in range(_SNB):
                        k = k0 + b

                        @pl.when(k < hi)
                        def _scat(b=b, k=k):
                            pltpu.make_async_copy(dsl(k), idx_bufs[b],
                                                  isems[b]).wait()
                            pltpu.make_async_copy(esl(k), val_bufs[b],
                                                  vsems[b]).wait()
                            pltpu.async_copy(val_bufs[b],
                                             shared.at[idx_bufs[b]],
                                             ssems[b], add=True)
                    for b in range(_SNB):
                        k = k0 + b

                        @pl.when(k < hi)
                        def _drain(b=b):
                            pltpu.make_async_copy(val_bufs[b],
                                                  shared.at[idx_bufs[b]],
                                                  ssems[b]).wait()
                    return carry

                nch = (hi - lo + _SNB - 1) // _SNB
                lax.fori_loop(0, nch, bod, 0)
                plsc.subcore_barrier()
                pltpu.sync_copy(
                    shared.at[pl.ds(r0, _ROWS_PER_TILE)],
                    ah.at[pl.ds(r0, _ROWS_PER_TILE),
                          pl.ds(c * _CHUNK, _CHUNK)])
                plsc.subcore_barrier()

    f = pl.kernel(
        body,
        out_type=jax.ShapeDtypeStruct((_N, _D), jnp.float32),
        mesh=mesh,
        compiler_params=pltpu.CompilerParams(use_tc_tiling_on_sc=False),
        scratch_types=(
            [pltpu.VMEM_SHARED((_N, _CHUNK), jnp.float32),
             pltpu.VMEM((125, _CHUNK), jnp.float32)]
            + [pltpu.VMEM((_GB,), jnp.int32) for _ in range(_SNB)]
            + [pltpu.VMEM((_GB, _CHUNK), jnp.float32) for _ in range(_SNB)]
            + [pltpu.SemaphoreType.DMA for _ in range(3 * _SNB)]
        ),
    )
    return f(enew, dst)


def kernel(node_x, body_edge_attr, body_edge_index, cable_edge_attr,
           cable_edge_index, contact_edge_attr, contact_edge_index, params):
    p = params
    x = _apply_block(p["enc_node"], [node_x])
    be = _apply_block(p["enc_body"], [body_edge_attr])
    ce = _apply_block(p["enc_cable"], [cable_edge_attr])
    cte = _apply_block(p["enc_contact"], [contact_edge_attr])

    sb, db = body_edge_index[0], body_edge_index[1]
    sc_, dc = cable_edge_index[0], cable_edge_index[1]
    sct, dct = contact_edge_index[0], contact_edge_index[1]

    for st in p["proc"]:
        xi_b, xj_b = _sc_gather(x, [db, sb])
        xi_c, xj_c = _sc_gather(x, [dc, sc_])
        xi_ct, xj_ct = _sc_gather(x, [dct, sct])
        be = _apply_block(st["body"], [xi_b, xj_b, be], resid=be)
        agg_b = _sc_scatter_one(be, db)
        ce = _apply_block(st["cable"], [xi_c, xj_c, ce], resid=ce)
        agg_c = _sc_scatter_one(ce, dc)
        cte = _apply_block(st["contact"], [xi_ct, xj_ct, cte], resid=cte)
        agg_ct = _sc_scatter_one(cte, dct)
        x = _apply_block(st["update"], [x, agg_b, agg_c, agg_ct], resid=x)

    dec = _apply_mlp(p["dec_node"], x)
    cdec = _apply_mlp(p["dec_cable"], ce)
    return (dec, cdec)


# speedup vs baseline: 1.0172x; 1.0021x over previous
import functools

import jax
import jax.numpy as jnp
from jax import lax
from jax.experimental import pallas as pl
from jax.experimental.pallas import tpu as pltpu
from jax.experimental.pallas import tpu_sc as plsc

_NC = 2
_NS = 16
_NW = _NC * _NS
_GB = 80

_N = 50000
_D = 128
_CHUNK = 32
_NCHUNK = _D // _CHUNK
_ROWS_PER_TILE = _N // _NS

_UNITS = (
    (0, 0, 0), (0, 1, 0), (0, 2, 0), (1, 0, 0), (1, 1, 0),
    (0, 3, 1), (2, 0, 1), (2, 1, 1), (2, 2, 1), (2, 3, 1), (1, 2, 1), (1, 3, 1),
)


def _tc_block_call(xs, Ws, b1, W2, b2, gamma, beta, resid, bs):
    R = xs[0].shape[0]
    H = W2.shape[0]
    O = W2.shape[1]
    nx = len(xs)
    ln = gamma is not None
    has_res = resid is not None

    def body(*refs):
        out_r = refs[-1]
        xr = refs[:nx]
        wr = refs[nx:2 * nx]
        k = 2 * nx
        b1r, w2r, b2r = refs[k], refs[k + 1], refs[k + 2]
        k += 3
        h = jnp.dot(xr[0][...], wr[0][...], preferred_element_type=jnp.float32)
        for t in range(1, nx):
            h = h + jnp.dot(xr[t][...], wr[t][...],
                            preferred_element_type=jnp.float32)
        h = jnp.maximum(h + b1r[...], 0.0)
        y = jnp.dot(h, w2r[...], preferred_element_type=jnp.float32) + b2r[...]
        if ln:
            gr, btr = refs[k], refs[k + 1]
            k += 2
            mu = jnp.mean(y, axis=-1, keepdims=True)
            d = y - mu
            var = jnp.mean(d * d, axis=-1, keepdims=True)
            y = gr[...] * (d / jnp.sqrt(var + 1e-5)) + btr[...]
        if has_res:
            y = y + refs[k][...]
        out_r[...] = y

    in_specs = (
        [pl.BlockSpec((bs, x.shape[1]), lambda i: (i, 0)) for x in xs]
        + [pl.BlockSpec(W.shape, lambda i: (0, 0)) for W in Ws]
        + [pl.BlockSpec((1, H), lambda i: (0, 0)),
           pl.BlockSpec((H, O), lambda i: (0, 0)),
           pl.BlockSpec((1, O), lambda i: (0, 0))]
    )
    args = list(xs) + list(Ws) + [b1.reshape(1, -1), W2, b2.reshape(1, -1)]
    if ln:
        in_specs += [pl.BlockSpec((1, O), lambda i: (0, 0))] * 2
        args += [gamma.reshape(1, -1), beta.reshape(1, -1)]
    if has_res:
        in_specs += [pl.BlockSpec((bs, O), lambda i: (i, 0))]
        args += [resid]
    return pl.pallas_call(
        body,
        grid=(R // bs,),
        in_specs=in_specs,
        out_specs=pl.BlockSpec((bs, O), lambda i: (i, 0)),
        out_shape=jax.ShapeDtypeStruct((R, O), jnp.float32),
    )(*args)


def _apply_block(blk, xs, resid=None, bs=2000):
    mlp = blk["mlp"]
    W1, b1 = mlp[0]["W"], mlp[0]["b"]
    W2, b2 = mlp[1]["W"], mlp[1]["b"]
    Ws, off = [], 0
    for xx in xs:
        w = xx.shape[1]
        Ws.append(lax.slice_in_dim(W1, off, off + w, axis=0).astype(xx.dtype))
        off += w
    return _tc_block_call(xs, Ws, b1, W2, b2, blk["gamma"], blk["beta"],
                          resid, bs)


def _apply_mlp(mlp, x, bs=2000):
    W1, b1 = mlp[0]["W"], mlp[0]["b"]
    W2, b2 = mlp[1]["W"], mlp[1]["b"]
    O = W2.shape[1]
    W2p = jnp.pad(W2, ((0, 0), (0, _D - O)))
    b2p = jnp.pad(b2, (0, _D - O))
    out = _tc_block_call([x], [W1], b1, W2p, b2p, None, None, None, bs)
    return out[:, :O]


_GNB = 8
_GBG = 80


def _sc_gather(x, idx_list):
    Es = [int(i.shape[0]) for i in idx_list]
    D = int(x.shape[1])
    dt = x.dtype
    na = len(Es)
    mesh = plsc.VectorSubcoreMesh(core_axis_name="c", subcore_axis_name="s")

    def body(*refs):
        x_hbm = refs[0]
        idx_refs = refs[1:1 + na]
        out_refs = refs[1 + na:1 + 2 * na]
        rest = refs[1 + 2 * na:]
        idx_bufs = rest[:_GNB]
        row_bufs = rest[_GNB:2 * _GNB]
        isems = rest[2 * _GNB:3 * _GNB]
        gsems = rest[3 * _GNB:4 * _GNB]
        wsems = rest[4 * _GNB:5 * _GNB]
        wid = lax.axis_index("s") * _NC + lax.axis_index("c")
        for a in range(na):
            nb = -(-Es[a] // _GBG)
            cnt = -(-nb // _NW)
            cnt = -(-cnt // _GNB) * _GNB
            span = nb - cnt
            start = (wid * span) // (_NW - 1)
            ih = idx_refs[a]
            oh = out_refs[a]

            def bod(g, carry, ih=ih, oh=oh, start=start, E=Es[a]):
                k0 = start + g * _GNB

                def bs(b):
                    return pl.multiple_of(
                        jnp.minimum((k0 + b) * _GBG, E - _GBG), 8)

                for b in range(_GNB):
                    pltpu.async_copy(ih.at[pl.ds(bs(b), _GBG)], idx_bufs[b],
                                     isems[b])
                for b in range(_GNB):
                    pltpu.make_async_copy(ih.at[pl.ds(bs(b), _GBG)],
                                          idx_bufs[b], isems[b]).wait()
                    pltpu.async_copy(x_hbm.at[idx_bufs[b]], row_bufs[b],
                                     gsems[b])
                for b in range(_GNB):
                    pltpu.make_async_copy(x_hbm.at[idx_bufs[b]], row_bufs[b],
                                          gsems[b]).wait()
                    pltpu.async_copy(row_bufs[b], oh.at[pl.ds(bs(b), _GBG)],
                                     wsems[b])
                for b in range(_GNB):
                    pltpu.make_async_copy(row_bufs[b],
                                          oh.at[pl.ds(bs(b), _GBG)],
                                          wsems[b]).wait()
                return carry

            lax.fori_loop(0, cnt // _GNB, bod, 0)

    f = pl.kernel(
        body,
        out_type=[jax.ShapeDtypeStruct((E, D), dt) for E in Es],
        mesh=mesh,
        compiler_params=pltpu.CompilerParams(use_tc_tiling_on_sc=False),
        scratch_types=(
            [pltpu.VMEM((_GBG,), jnp.int32) for _ in range(_GNB)]
            + [pltpu.VMEM((_GBG, D), dt) for _ in range(_GNB)]
            + [pltpu.SemaphoreType.DMA for _ in range(3 * _GNB)]
        ),
    )
    return f(x, *idx_list)


_SNB = 6


_SNB = 6


def _sc_scatter_one(enew, dst):
    units = ((0, 0), (1, 0), (2, 1), (3, 1))
    E = int(enew.shape[0])
    mesh = plsc.VectorSubcoreMesh(core_axis_name="c", subcore_axis_name="s")

    def body(*refs):
        eh, dh, ah, shared, zbuf = refs[:5]
        rest = refs[5:]
        idx_bufs = rest[:_SNB]
        val_bufs = rest[_SNB:2 * _SNB]
        isems = rest[2 * _SNB:3 * _SNB]
        vsems = rest[3 * _SNB:4 * _SNB]
        ssems = rest[4 * _SNB:5 * _SNB]
        cid = lax.axis_index("c")
        sid = lax.axis_index("s")

        def zinit(r, carry):
            zbuf[r, pl.ds(0, 16)] = jnp.zeros((16,), jnp.float32)
            zbuf[r, pl.ds(16, 16)] = jnp.zeros((16,), jnp.float32)
            return carry

        lax.fori_loop(0, 125, zinit, 0)

        r0 = sid * _ROWS_PER_TILE
        nb = E // _GB
        lo = (nb * sid) // _NS
        hi = (nb * (sid + 1)) // _NS
        for (c, core) in units:
            @pl.when(cid == core)
            def _unit(c=c):
                for j in range(_ROWS_PER_TILE // 125):
                    pltpu.async_copy(zbuf,
                                     shared.at[pl.ds(r0 + j * 125, 125)],
                                     isems[0])
                for j in range(_ROWS_PER_TILE // 125):
                    pltpu.make_async_copy(
                        zbuf, shared.at[pl.ds(r0 + j * 125, 125)],
                        isems[0]).wait()
                plsc.subcore_barrier()

                def esl(k, c=c):
                    base = pl.multiple_of(k * _GB, 8)
                    return eh.at[pl.ds(base, _GB),
                                 pl.ds(c * _CHUNK, _CHUNK)]

                def dsl(k):
                    base = pl.multiple_of(k * _GB, 8)
                    return dh.at[pl.ds(base, _GB)]

                def bod(g, carry):
                    k0 = lo + g * _SNB
                    for b in range(_SNB):
                        k = k0 + b

                        @pl.when(k < hi)
                        def _fire(b=b, k=k):
                            pltpu.async_copy(dsl(k), idx_bufs[b], isems[b])
                            pltpu.async_copy(esl(k), val_bufs[b], vsems[b])
                    for b in range(_SNB):
                        k = k0 + b

                        @pl.when(k < hi)
                        def _scat(b=b, k=k):
                            pltpu.make_async_copy(dsl(k), idx_bufs[b],
                                                  isems[b]).wait()
                            pltpu.make_async_copy(esl(k), val_bufs[b],
                                                  vsems[b]).wait()
                            pltpu.async_copy(val_bufs[b],
                                             shared.at[idx_bufs[b]],
                                             ssems[b], add=True)
                    for b in range(_SNB):
                        k = k0 + b

                        @pl.when(k < hi)
                        def _drain(b=b):
                            pltpu.make_async_copy(val_bufs[b],
                                                  shared.at[idx_bufs[b]],
                                                  ssems[b]).wait()
                    return carry

                nch = (hi - lo + _SNB - 1) // _SNB
                lax.fori_loop(0, nch, bod, 0)
                plsc.subcore_barrier()
                pltpu.sync_copy(
                    shared.at[pl.ds(r0, _ROWS_PER_TILE)],
                    ah.at[pl.ds(r0, _ROWS_PER_TILE),
                          pl.ds(c * _CHUNK, _CHUNK)])
                plsc.subcore_barrier()

    f = pl.kernel(
        body,
        out_type=jax.ShapeDtypeStruct((_N, _D), jnp.float32),
        mesh=mesh,
        compiler_params=pltpu.CompilerParams(use_tc_tiling_on_sc=False),
        scratch_types=(
            [pltpu.VMEM_SHARED((_N, _CHUNK), jnp.float32),
             pltpu.VMEM((125, _CHUNK), jnp.float32)]
            + [pltpu.VMEM((_GB,), jnp.int32) for _ in range(_SNB)]
            + [pltpu.VMEM((_GB, _CHUNK), jnp.float32) for _ in range(_SNB)]
            + [pltpu.SemaphoreType.DMA for _ in range(3 * _SNB)]
        ),
    )
    return f(enew, dst)


def kernel(node_x, body_edge_attr, body_edge_index, cable_edge_attr,
           cable_edge_index, contact_edge_attr, contact_edge_index, params):
    p = params
    x = _apply_block(p["enc_node"], [node_x])
    be = _apply_block(p["enc_body"], [body_edge_attr])
    ce = _apply_block(p["enc_cable"], [cable_edge_attr])
    cte = _apply_block(p["enc_contact"], [contact_edge_attr])

    sb, db = body_edge_index[0], body_edge_index[1]
    sc_, dc = cable_edge_index[0], cable_edge_index[1]
    sct, dct = contact_edge_index[0], contact_edge_index[1]

    for st in p["proc"]:
        xi_b, xj_b = _sc_gather(x, [db, sb])
        xi_c, xj_c = _sc_gather(x, [dc, sc_])
        xi_ct, xj_ct = _sc_gather(x, [dct, sct])
        be = _apply_block(st["body"], [xi_b, xj_b, be], resid=be)
        agg_b = _sc_scatter_one(be, db)
        ce = _apply_block(st["cable"], [xi_c, xj_c, ce], resid=ce)
        agg_c = _sc_scatter_one(ce, dc)
        cte = _apply_block(st["contact"], [xi_ct, xj_ct, cte], resid=cte)
        agg_ct = _sc_scatter_one(cte, dct)
        x = _apply_block(st["update"], [x, agg_b, agg_c, agg_ct], resid=x)

    dec = _apply_mlp(p["dec_node"], x)
    cdec = _apply_mlp(p["dec_cable"], ce)
    return (dec, cdec)
